# baseline (speedup 1.0000x reference)
import functools

import jax
import jax.numpy as jnp
from jax import lax
from jax.experimental import pallas as pl
from jax.experimental.pallas import tpu as pltpu
from jax.experimental.pallas import tpu_sc as plsc


@functools.cache
def _sc_gather(vocab, d, batch):
    info = plsc.get_sparse_core_info()
    nw = info.num_cores * info.num_subcores
    assert batch % (8 * nw) == 0 and d % info.num_lanes == 0
    b_per_w = batch // nw

    mesh = plsc.VectorSubcoreMesh(core_axis_name="c", subcore_axis_name="s")

    @functools.partial(
        pl.kernel,
        mesh=mesh,
        out_type=jax.ShapeDtypeStruct((batch, d), jnp.float32),
        scratch_types=[
            pltpu.VMEM((b_per_w,), jnp.int32),
            pltpu.VMEM((b_per_w, d), jnp.float32),
            pltpu.SemaphoreType.DMA,
        ],
        compiler_params=pltpu.CompilerParams(use_tc_tiling_on_sc=False),
    )
    def gather(table_hbm, idx_hbm, out_hbm, idx_v, rows_v, sem):
        wid = lax.axis_index("s") * info.num_cores + lax.axis_index("c")
        base = wid * b_per_w
        pltpu.sync_copy(idx_hbm.at[pl.ds(base, b_per_w)], idx_v)
        pltpu.async_copy(table_hbm.at[idx_v], rows_v, sem).wait()
        pltpu.sync_copy(rows_v, out_hbm.at[pl.ds(base, b_per_w)])

    return gather


def _ln(x, g, b, eps=1e-5):
    m = jnp.mean(x, axis=-1, keepdims=True)
    c = x - m
    v = jnp.mean(c * c, axis=-1, keepdims=True)
    return c * lax.rsqrt(v + eps) * g + b


def _head_body(x_ref, p_ref, w_ref, bias_ref, o_ref):
    x = x_ref[...]
    p = p_ref[...]
    x = _ln(x, p[0:1, :], p[1:2, :])
    x = _ln(x, p[2:3, :], p[3:4, :])
    x = _ln(x, p[4:5, :], p[5:6, :])
    o_ref[...] = (
        jnp.dot(x, w_ref[...], preferred_element_type=jnp.float32) + bias_ref[...]
    )


@functools.cache
def _head(batch, d, vocab_out, block_rows):
    grid = batch // block_rows
    return pl.pallas_call(
        _head_body,
        grid=(grid,),
        in_specs=[
            pl.BlockSpec((block_rows, d), lambda i: (i, 0)),
            pl.BlockSpec((6, d), lambda i: (0, 0)),
            pl.BlockSpec((d, vocab_out), lambda i: (0, 0)),
            pl.BlockSpec((1, vocab_out), lambda i: (0, 0)),
        ],
        out_specs=pl.BlockSpec((block_rows, vocab_out), lambda i: (i, 0)),
        out_shape=jax.ShapeDtypeStruct((batch, vocab_out), jnp.float32),
    )


def kernel(indices, emb, g0, b0, g1, b1, gf, bf, W, bias):
    vocab, d = emb.shape
    vocab_out = W.shape[1]
    idx = indices.reshape(-1).astype(jnp.int32)
    batch = idx.shape[0]

    gathered = _sc_gather(vocab, d, batch)(emb, idx)
    params = jnp.stack([g0, b0, g1, b1, gf, bf], axis=0)
    out = _head(batch, d, vocab_out, 256)(
        gathered, params, W, bias.reshape(1, vocab_out)
    )
    return out.reshape(indices.shape + (vocab_out,))


# speedup vs baseline: 1.0193x; 1.0193x over previous
import functools

import jax
import jax.numpy as jnp
from jax import lax
from jax.experimental import pallas as pl
from jax.experimental.pallas import tpu as pltpu
from jax.experimental.pallas import tpu_sc as plsc


@functools.cache
def _sc_gather(vocab, dpad, batch, chunks):
    info = plsc.get_sparse_core_info()
    nw = info.num_cores * info.num_subcores
    assert batch % (8 * nw) == 0 and dpad % 128 == 0
    b_per_w = batch // nw
    assert b_per_w % chunks == 0
    b_chunk = b_per_w // chunks

    mesh = plsc.VectorSubcoreMesh(core_axis_name="c", subcore_axis_name="s")

    @functools.partial(
        pl.kernel,
        mesh=mesh,
        out_type=jax.ShapeDtypeStruct((batch, dpad), jnp.float32),
        scratch_types=[
            pltpu.VMEM((b_per_w,), jnp.int32),
            pltpu.VMEM((b_chunk, dpad), jnp.float32),
            pltpu.SemaphoreType.DMA,
        ],
    )
    def gather(table_hbm, idx_hbm, out_hbm, idx_v, rows_v, sem):
        wid = lax.axis_index("s") * info.num_cores + lax.axis_index("c")
        base = wid * b_per_w
        pltpu.sync_copy(idx_hbm.at[pl.ds(base, b_per_w)], idx_v)
        for c in range(chunks):
            pltpu.async_copy(
                table_hbm.at[idx_v.at[pl.ds(c * b_chunk, b_chunk)]], rows_v, sem
            ).wait()
            pltpu.sync_copy(rows_v, out_hbm.at[pl.ds(base + c * b_chunk, b_chunk)])

    return gather


def _ln(x, g, b, eps=1e-5):
    m = jnp.mean(x, axis=-1, keepdims=True)
    c = x - m
    v = jnp.mean(c * c, axis=-1, keepdims=True)
    return c * lax.rsqrt(v + eps) * g + b


def _head_body(x_ref, p_ref, w_ref, bias_ref, o_ref):
    d = p_ref.shape[1]
    x = x_ref[:, :d]
    p = p_ref[...]
    x = _ln(x, p[0:1, :], p[1:2, :])
    x = _ln(x, p[2:3, :], p[3:4, :])
    x = _ln(x, p[4:5, :], p[5:6, :])
    o_ref[...] = (
        jnp.dot(x, w_ref[...], preferred_element_type=jnp.float32) + bias_ref[...]
    )


@functools.cache
def _head(batch, dpad, d, vocab_out, block_rows):
    grid = batch // block_rows
    return pl.pallas_call(
        _head_body,
        grid=(grid,),
        in_specs=[
            pl.BlockSpec((block_rows, dpad), lambda i: (i, 0)),
            pl.BlockSpec((6, d), lambda i: (0, 0)),
            pl.BlockSpec((d, vocab_out), lambda i: (0, 0)),
            pl.BlockSpec((1, vocab_out), lambda i: (0, 0)),
        ],
        out_specs=pl.BlockSpec((block_rows, vocab_out), lambda i: (i, 0)),
        out_shape=jax.ShapeDtypeStruct((batch, vocab_out), jnp.float32),
    )


def kernel(indices, emb, g0, b0, g1, b1, gf, bf, W, bias):
    vocab, d = emb.shape
    vocab_out = W.shape[1]
    idx = indices.reshape(-1).astype(jnp.int32)
    batch = idx.shape[0]
    dpad = 128
    emb_pad = jnp.pad(emb, ((0, 0), (0, dpad - d)))

    gathered = _sc_gather(vocab, dpad, batch, 2)(emb_pad, idx)
    params = jnp.stack([g0, b0, g1, b1, gf, bf], axis=0)
    out = _head(batch, dpad, d, vocab_out, 256)(
        gathered, params, W, bias.reshape(1, vocab_out)
    )
    return out.reshape(indices.shape + (vocab_out,))
